# Initial kernel scaffold; baseline (speedup 1.0000x reference)
import jax
import jax.numpy as jnp
from jax.experimental import pallas as pl


def kernel(x, indices, W, b):
    raise NotImplementedError("write your pallas kernel here")


# speedup vs baseline: 2.6076x; 2.6076x over previous
import functools

import jax
import jax.numpy as jnp
from jax import lax
from jax.experimental import pallas as pl
from jax.experimental.pallas import tpu as pltpu
from jax.experimental.pallas import tpu_sc as plsc

N_NODES = 10000
SEQ_LEN = 32
IN_CH = 128
OUT_CH = 128

NUM_CORES = 2
NUM_SUBCORES = 16
NUM_WORKERS = NUM_CORES * NUM_SUBCORES
TOTAL_ROWS = N_NODES * SEQ_LEN
ROWS_PER_WORKER = TOTAL_ROWS // NUM_WORKERS
CHUNK = 1000


def _sc_gather_body(table_hbm, idx_hbm, out_hbm, idx_v, rows_v, sem):
    wid = lax.axis_index("s") * NUM_CORES + lax.axis_index("c")
    base = wid * ROWS_PER_WORKER

    def body(c, carry):
        off = base + c * CHUNK
        pltpu.sync_copy(idx_hbm.at[pl.ds(off, CHUNK)], idx_v)
        pltpu.async_copy(table_hbm.at[idx_v], rows_v, sem).wait()
        pltpu.sync_copy(rows_v, out_hbm.at[pl.ds(off, CHUNK)])
        return carry

    lax.fori_loop(0, ROWS_PER_WORKER // CHUNK, body, 0)


def _sc_gather(x, idx_flat):
    mesh = plsc.VectorSubcoreMesh(core_axis_name="c", subcore_axis_name="s")
    kfn = pl.kernel(
        _sc_gather_body,
        mesh=mesh,
        out_type=jax.ShapeDtypeStruct((TOTAL_ROWS, IN_CH), jnp.float32),
        scratch_types=[
            pltpu.VMEM((CHUNK,), jnp.int32),
            pltpu.VMEM((CHUNK, IN_CH), jnp.float32),
            pltpu.SemaphoreType.DMA,
        ],
    )
    return kfn(x, idx_flat)


def _mm_body(g_ref, w_ref, b_ref, o_ref):
    o_ref[...] = (
        lax.dot_general(
            g_ref[...], w_ref[...],
            (((1,), (1,)), ((), ())),
            preferred_element_type=jnp.float32,
        )
        + b_ref[...]
    )


def _tc_matmul(g, W, b):
    m_block = 400
    grid = (N_NODES // m_block,)
    return pl.pallas_call(
        _mm_body,
        grid=grid,
        in_specs=[
            pl.BlockSpec((m_block, SEQ_LEN * IN_CH), lambda i: (i, 0)),
            pl.BlockSpec((OUT_CH, SEQ_LEN * IN_CH), lambda i: (0, 0)),
            pl.BlockSpec((1, OUT_CH), lambda i: (0, 0)),
        ],
        out_specs=pl.BlockSpec((m_block, OUT_CH), lambda i: (i, 0)),
        out_shape=jax.ShapeDtypeStruct((N_NODES, OUT_CH), jnp.float32),
    )(g, W, b)


@jax.jit
def kernel(x, indices, W, b):
    idx_flat = indices.reshape(-1).astype(jnp.int32)
    gathered = _sc_gather(x, idx_flat)
    g = gathered.reshape(N_NODES, SEQ_LEN * IN_CH)
    return _tc_matmul(g, W, b.reshape(1, OUT_CH))


# speedup vs baseline: 2.6220x; 1.0056x over previous
import functools

import jax
import jax.numpy as jnp
from jax import lax
from jax.experimental import pallas as pl
from jax.experimental.pallas import tpu as pltpu
from jax.experimental.pallas import tpu_sc as plsc

N_NODES = 10000
SEQ_LEN = 32
IN_CH = 128
OUT_CH = 128

NUM_CORES = 2
NUM_SUBCORES = 16
NUM_WORKERS = NUM_CORES * NUM_SUBCORES
TOTAL_ROWS = N_NODES * SEQ_LEN
ROWS_PER_WORKER = TOTAL_ROWS // NUM_WORKERS
CHUNK = 400
N_CHUNKS = ROWS_PER_WORKER // CHUNK


def _sc_gather_body(table_hbm, idx_hbm, out_hbm,
                    idx_v0, idx_v1, rows_v0, rows_v1,
                    gsem0, gsem1, wsem0, wsem1):
    wid = lax.axis_index("s") * NUM_CORES + lax.axis_index("c")
    base = wid * ROWS_PER_WORKER
    idx_v = (idx_v0, idx_v1)
    rows_v = (rows_v0, rows_v1)
    gsem = (gsem0, gsem1)
    wsem = (wsem0, wsem1)

    def start_gather(c):
        b = c % 2
        pltpu.sync_copy(idx_hbm.at[pl.ds(base + c * CHUNK, CHUNK)], idx_v[b])
        pltpu.make_async_copy(table_hbm.at[idx_v[b]], rows_v[b], gsem[b]).start()

    start_gather(0)
    start_gather(1)
    for c in range(N_CHUNKS):
        b = c % 2
        pltpu.make_async_copy(table_hbm.at[idx_v[b]], rows_v[b], gsem[b]).wait()
        wb = pltpu.make_async_copy(
            rows_v[b], out_hbm.at[pl.ds(base + c * CHUNK, CHUNK)], wsem[b])
        wb.start()
        if c + 2 < N_CHUNKS:
            wb.wait()
            start_gather(c + 2)
        else:
            wb.wait()


def _sc_gather(x_bf, idx_flat):
    mesh = plsc.VectorSubcoreMesh(core_axis_name="c", subcore_axis_name="s")
    kfn = pl.kernel(
        _sc_gather_body,
        mesh=mesh,
        out_type=jax.ShapeDtypeStruct((TOTAL_ROWS, IN_CH), jnp.float32),
        scratch_types=[
            pltpu.VMEM((CHUNK,), jnp.int32),
            pltpu.VMEM((CHUNK,), jnp.int32),
            pltpu.VMEM((CHUNK, IN_CH), jnp.float32),
            pltpu.VMEM((CHUNK, IN_CH), jnp.float32),
            pltpu.SemaphoreType.DMA,
            pltpu.SemaphoreType.DMA,
            pltpu.SemaphoreType.DMA,
            pltpu.SemaphoreType.DMA,
        ],
    )
    return kfn(x_bf, idx_flat)


def _mm_body(g_ref, w_ref, b_ref, o_ref):
    o_ref[...] = (
        lax.dot_general(
            g_ref[...].astype(jnp.bfloat16), w_ref[...].astype(jnp.bfloat16),
            (((1,), (1,)), ((), ())),
            preferred_element_type=jnp.float32,
        )
        + b_ref[...]
    )


def _tc_matmul(g, W, b):
    m_block = 400
    grid = (N_NODES // m_block,)
    return pl.pallas_call(
        _mm_body,
        grid=grid,
        in_specs=[
            pl.BlockSpec((m_block, SEQ_LEN * IN_CH), lambda i: (i, 0)),
            pl.BlockSpec((OUT_CH, SEQ_LEN * IN_CH), lambda i: (0, 0)),
            pl.BlockSpec((1, OUT_CH), lambda i: (0, 0)),
        ],
        out_specs=pl.BlockSpec((m_block, OUT_CH), lambda i: (i, 0)),
        out_shape=jax.ShapeDtypeStruct((N_NODES, OUT_CH), jnp.float32),
    )(g, W, b)


@jax.jit
def kernel(x, indices, W, b):
    idx_flat = indices.reshape(-1).astype(jnp.int32)
    gathered = _sc_gather(x, idx_flat)
    g = gathered.reshape(N_NODES, SEQ_LEN * IN_CH)
    return _tc_matmul(g, W, b.reshape(1, OUT_CH))


# speedup vs baseline: 3.3483x; 1.2770x over previous
import functools

import jax
import jax.numpy as jnp
from jax import lax
from jax.experimental import pallas as pl
from jax.experimental.pallas import tpu as pltpu
from jax.experimental.pallas import tpu_sc as plsc

N_NODES = 10000
SEQ_LEN = 32
IN_CH = 128
OUT_CH = 128

NUM_CORES = 2
NUM_SUBCORES = 16
NUM_WORKERS = NUM_CORES * NUM_SUBCORES
TOTAL_ROWS = N_NODES * SEQ_LEN
ROWS_PER_WORKER = TOTAL_ROWS // NUM_WORKERS
CHUNK = 400
N_CHUNKS = ROWS_PER_WORKER // CHUNK


def _sc_gather_body(table_hbm, idx_hbm, out_hbm,
                    idx_v0, idx_v1, rows_v0, rows_v1,
                    gsem0, gsem1, wsem0, wsem1):
    wid = lax.axis_index("s") * NUM_CORES + lax.axis_index("c")
    base = wid * ROWS_PER_WORKER
    idx_v = (idx_v0, idx_v1)
    rows_v = (rows_v0, rows_v1)
    gsem = (gsem0, gsem1)
    wsem = (wsem0, wsem1)

    def start_gather(c):
        b = c % 2
        pltpu.sync_copy(idx_hbm.at[pl.ds(base + c * CHUNK, CHUNK)], idx_v[b])
        pltpu.make_async_copy(table_hbm.at[idx_v[b]], rows_v[b], gsem[b]).start()

    start_gather(0)
    start_gather(1)
    for c in range(N_CHUNKS):
        b = c % 2
        pltpu.make_async_copy(table_hbm.at[idx_v[b]], rows_v[b], gsem[b]).wait()
        wb = pltpu.make_async_copy(
            rows_v[b], out_hbm.at[pl.ds(base + c * CHUNK, CHUNK)], wsem[b])
        wb.start()
        if c + 2 < N_CHUNKS:
            wb.wait()
            start_gather(c + 2)
        else:
            wb.wait()


def _sc_gather(x, idx_flat):
    mesh = plsc.VectorSubcoreMesh(core_axis_name="c", subcore_axis_name="s")
    kfn = pl.kernel(
        _sc_gather_body,
        mesh=mesh,
        out_type=jax.ShapeDtypeStruct((TOTAL_ROWS, IN_CH), jnp.float32),
        scratch_types=[
            pltpu.VMEM((CHUNK,), jnp.int32),
            pltpu.VMEM((CHUNK,), jnp.int32),
            pltpu.VMEM((CHUNK, IN_CH), jnp.float32),
            pltpu.VMEM((CHUNK, IN_CH), jnp.float32),
            pltpu.SemaphoreType.DMA,
            pltpu.SemaphoreType.DMA,
            pltpu.SemaphoreType.DMA,
            pltpu.SemaphoreType.DMA,
        ],
    )
    return kfn(x, idx_flat)


def _mm_body(g_ref, wt_ref, b_ref, o_ref):
    s = pl.program_id(1)

    @pl.when(s == 0)
    def _init():
        o_ref[...] = jnp.broadcast_to(b_ref[...], o_ref.shape)

    o_ref[...] += lax.dot_general(
        g_ref[0].astype(jnp.bfloat16), wt_ref[0].astype(jnp.bfloat16),
        (((1,), (0,)), ((), ())),
        preferred_element_type=jnp.float32,
    )


def _tc_matmul(gout, Wt, b):
    m_block = 2000
    grid = (N_NODES // m_block, SEQ_LEN)
    return pl.pallas_call(
        _mm_body,
        grid=grid,
        in_specs=[
            pl.BlockSpec((1, m_block, IN_CH), lambda i, s: (s, i, 0)),
            pl.BlockSpec((1, IN_CH, OUT_CH), lambda i, s: (s, 0, 0)),
            pl.BlockSpec((1, OUT_CH), lambda i, s: (0, 0)),
        ],
        out_specs=pl.BlockSpec((m_block, OUT_CH), lambda i, s: (i, 0)),
        out_shape=jax.ShapeDtypeStruct((N_NODES, OUT_CH), jnp.float32),
    )(gout, Wt, b)


@jax.jit
def kernel(x, indices, W, b):
    idx_flat = indices.astype(jnp.int32).T.reshape(-1)
    gathered = _sc_gather(x, idx_flat)
    gout = gathered.reshape(SEQ_LEN, N_NODES, IN_CH)
    Wt = W.reshape(OUT_CH, SEQ_LEN, IN_CH).transpose(1, 2, 0)
    return _tc_matmul(gout, Wt, b.reshape(1, OUT_CH))


# speedup vs baseline: 4.7349x; 1.4141x over previous
import functools

import jax
import jax.numpy as jnp
from jax import lax
from jax.experimental import pallas as pl
from jax.experimental.pallas import tpu as pltpu
from jax.experimental.pallas import tpu_sc as plsc

N_NODES = 10000
SEQ_LEN = 32
IN_CH = 128
OUT_CH = 128

NUM_CORES = 2
NUM_SUBCORES = 16
NUM_WORKERS = NUM_CORES * NUM_SUBCORES
TOTAL_ROWS = N_NODES * SEQ_LEN
ROWS_PER_WORKER = TOTAL_ROWS // NUM_WORKERS
CHUNK = 400
N_CHUNKS = ROWS_PER_WORKER // CHUNK


def _sc_gather_body(table_hbm, idx_hbm, out_hbm,
                    idx_v0, idx_v1, rows_v0, rows_v1,
                    gsem0, gsem1, wsem0, wsem1):
    wid = lax.axis_index("s") * NUM_CORES + lax.axis_index("c")
    base = wid * ROWS_PER_WORKER
    idx_v = (idx_v0, idx_v1)
    rows_v = (rows_v0, rows_v1)
    gsem = (gsem0, gsem1)
    wsem = (wsem0, wsem1)

    def start_gather(c):
        b = c % 2
        pltpu.sync_copy(idx_hbm.at[pl.ds(base + c * CHUNK, CHUNK)], idx_v[b])
        pltpu.make_async_copy(table_hbm.at[idx_v[b]], rows_v[b], gsem[b]).start()

    start_gather(0)
    start_gather(1)
    for c in range(N_CHUNKS):
        b = c % 2
        pltpu.make_async_copy(table_hbm.at[idx_v[b]], rows_v[b], gsem[b]).wait()
        wb = pltpu.make_async_copy(
            rows_v[b], out_hbm.at[pl.ds(base + c * CHUNK, CHUNK)], wsem[b])
        wb.start()
        if c + 2 < N_CHUNKS:
            wb.wait()
            start_gather(c + 2)
        else:
            wb.wait()


def _sc_gather(x, idx_flat):
    mesh = plsc.VectorSubcoreMesh(core_axis_name="c", subcore_axis_name="s")
    kfn = pl.kernel(
        _sc_gather_body,
        mesh=mesh,
        out_type=jax.ShapeDtypeStruct((TOTAL_ROWS, IN_CH), jnp.float32),
        scratch_types=[
            pltpu.VMEM((CHUNK,), jnp.int32),
            pltpu.VMEM((CHUNK,), jnp.int32),
            pltpu.VMEM((CHUNK, IN_CH), jnp.float32),
            pltpu.VMEM((CHUNK, IN_CH), jnp.float32),
            pltpu.SemaphoreType.DMA,
            pltpu.SemaphoreType.DMA,
            pltpu.SemaphoreType.DMA,
            pltpu.SemaphoreType.DMA,
        ],
    )
    return kfn(x, idx_flat)


def _mm_body(g_ref, wt_ref, b_ref, o_ref):
    acc = jnp.broadcast_to(b_ref[...], o_ref.shape)
    for s in range(SEQ_LEN):
        acc = acc + lax.dot_general(
            g_ref[s].astype(jnp.bfloat16), wt_ref[s].astype(jnp.bfloat16),
            (((1,), (0,)), ((), ())),
            preferred_element_type=jnp.float32,
        )
    o_ref[...] = acc


def _tc_matmul(gout, Wt, b):
    m_block = 1000
    grid = (N_NODES // m_block,)
    return pl.pallas_call(
        _mm_body,
        grid=grid,
        in_specs=[
            pl.BlockSpec((SEQ_LEN, m_block, IN_CH), lambda i: (0, i, 0)),
            pl.BlockSpec((SEQ_LEN, IN_CH, OUT_CH), lambda i: (0, 0, 0)),
            pl.BlockSpec((1, OUT_CH), lambda i: (0, 0)),
        ],
        out_specs=pl.BlockSpec((m_block, OUT_CH), lambda i: (i, 0)),
        out_shape=jax.ShapeDtypeStruct((N_NODES, OUT_CH), jnp.float32),
    )(gout, Wt, b)


@jax.jit
def kernel(x, indices, W, b):
    idx_flat = indices.astype(jnp.int32).T.reshape(-1)
    gathered = _sc_gather(x, idx_flat)
    gout = gathered.reshape(SEQ_LEN, N_NODES, IN_CH)
    Wt = W.reshape(OUT_CH, SEQ_LEN, IN_CH).transpose(1, 2, 0)
    return _tc_matmul(gout, Wt, b.reshape(1, OUT_CH))
